# Initial kernel scaffold; baseline (speedup 1.0000x reference)
import jax
import jax.numpy as jnp
from jax.experimental import pallas as pl


def kernel(z, emb_weight):
    raise NotImplementedError("write your pallas kernel here")


# speedup vs baseline: 1.0361x; 1.0361x over previous
import functools

import jax
import jax.numpy as jnp
from jax import lax
from jax.experimental import pallas as pl
from jax.experimental.pallas import tpu as pltpu
from jax.experimental.pallas import tpu_sc as plsc

KK = 8192
DD = 32
NN = 8192
BN = 256
NB = NN // BN

NUM_WORKERS = 32
CH = NN // NUM_WORKERS


def _argmin_body(x_ref, sz_ref, w_ref, sw_ref, tok_ref):
    x = x_ref[...]
    w = w_ref[...]
    sz = sz_ref[...]
    sw = sw_ref[...]
    mm = lax.dot_general(x, w, (((1,), (1,)), ((), ())),
                         preferred_element_type=jnp.float32)
    d = (sz + sw) - 2.0 * mm
    minv = jnp.min(d, axis=1, keepdims=True)
    ii = lax.broadcasted_iota(jnp.int32, (BN, KK), 1)
    idx = jnp.min(jnp.where(d == minv, ii, jnp.int32(KK)), axis=1)
    tok_ref[0, 0, :] = idx


def _tokens(zf, sz, w, sw):
    tok3 = pl.pallas_call(
        _argmin_body,
        grid=(NB,),
        in_specs=[
            pl.BlockSpec((BN, DD), lambda i: (i, 0)),
            pl.BlockSpec((BN, 1), lambda i: (i, 0)),
            pl.BlockSpec((KK, DD), lambda i: (0, 0)),
            pl.BlockSpec((1, KK), lambda i: (0, 0)),
        ],
        out_specs=pl.BlockSpec((1, 1, BN), lambda i: (i, 0, 0)),
        out_shape=jax.ShapeDtypeStruct((NB, 1, BN), jnp.int32),
    )(zf, sz, w, sw)
    return tok3.reshape(NN)


def _fill(ref, rows, value):
    def body(i, carry):
        ref[i, :] = jnp.full((16,), value, jnp.float32)
        return carry
    lax.fori_loop(0, rows, body, 0)


def _sc_body(emb_hbm, tok_hbm, zq_hbm, hist_hbm,
             idx_v, rows_v, const_v, shared_hist, sem):
    cid = lax.axis_index("c")
    sid = lax.axis_index("s")
    wid = sid * 2 + cid
    base = wid * CH
    pltpu.sync_copy(tok_hbm.at[pl.ds(base, CH)], idx_v)
    pltpu.async_copy(emb_hbm.at[idx_v], rows_v, sem).wait()
    pltpu.sync_copy(rows_v, zq_hbm.at[pl.ds(base, CH)])
    zbase = sid * (KK // 16)
    _fill(const_v, CH, 0.0)
    pltpu.sync_copy(const_v, shared_hist.at[pl.ds(zbase, CH)])
    pltpu.sync_copy(const_v, shared_hist.at[pl.ds(zbase + CH, CH)])
    plsc.subcore_barrier()
    _fill(const_v, CH, 1.0)
    pltpu.sync_copy(const_v, shared_hist.at[idx_v], add=True)
    plsc.subcore_barrier()
    pltpu.sync_copy(shared_hist.at[pl.ds(zbase, CH)],
                    hist_hbm.at[cid, pl.ds(zbase, CH)])
    pltpu.sync_copy(shared_hist.at[pl.ds(zbase + CH, CH)],
                    hist_hbm.at[cid, pl.ds(zbase + CH, CH)])


def _gather_hist(emb, token):
    mesh = plsc.VectorSubcoreMesh(core_axis_name="c", subcore_axis_name="s")
    f = functools.partial(
        pl.kernel,
        mesh=mesh,
        compiler_params=pltpu.CompilerParams(use_tc_tiling_on_sc=False),
        out_type=[
            jax.ShapeDtypeStruct((NN, DD), jnp.float32),
            jax.ShapeDtypeStruct((2, KK, 16), jnp.float32),
        ],
        scratch_types=[
            pltpu.VMEM((CH,), jnp.int32),
            pltpu.VMEM((CH, DD), jnp.float32),
            pltpu.VMEM((CH, 16), jnp.float32),
            pltpu.VMEM_SHARED((KK, 16), jnp.float32),
            pltpu.SemaphoreType.DMA,
        ],
    )(_sc_body)
    return f(emb, token)


def _stats_body(zf_ref, zq_ref, hist_ref, st_ref,
                loss_ref, qe_ref, util_ref, perp_ref):
    zf = zf_ref[...]
    zq = zq_ref[...]
    dsq = (zq - zf) ** 2
    s = jnp.sum(dsq)
    m = s / jnp.float32(NN * DD)
    loss_ref[...] = jnp.reshape(0.25 * m + m, (1, 1))
    qe_ref[...] = jnp.reshape(s / jnp.float32(NN), (1, 1))
    h = hist_ref[0, :, 0:1] + hist_ref[1, :, 0:1]
    util_ref[...] = jnp.reshape(
        jnp.sum((h > 0).astype(jnp.float32)) / jnp.float32(KK), (1, 1))
    p = h / jnp.sum(h)
    perp_ref[...] = jnp.reshape(
        jnp.exp(-jnp.sum(p * jnp.log(p + 1e-10))), (1, 1))
    st_ref[...] = zf + (zq - zf)


def _stats(zf, zq, hist):
    return pl.pallas_call(
        _stats_body,
        out_shape=[
            jax.ShapeDtypeStruct((NN, DD), jnp.float32),
            jax.ShapeDtypeStruct((1, 1), jnp.float32),
            jax.ShapeDtypeStruct((1, 1), jnp.float32),
            jax.ShapeDtypeStruct((1, 1), jnp.float32),
            jax.ShapeDtypeStruct((1, 1), jnp.float32),
        ],
    )(zf, zq, hist)


def kernel(z, emb_weight):
    zp = jnp.transpose(z, (0, 2, 3, 1))
    zf = zp.reshape(NN, DD)
    sz = jnp.sum(zf ** 2, axis=1, keepdims=True)
    sw = jnp.sum(emb_weight ** 2, axis=1)
    token = _tokens(zf, sz, emb_weight, sw.reshape(1, KK))
    zq, hist = _gather_hist(emb_weight, token)
    st, loss, qe, util, perp = _stats(zf, zq, hist)
    out = jnp.transpose(st.reshape(zp.shape), (0, 3, 1, 2))
    return (out, loss[0, 0], qe[0, 0], util[0, 0], perp[0, 0])


# speedup vs baseline: 1.0686x; 1.0314x over previous
import functools

import jax
import jax.numpy as jnp
from jax import lax
from jax.experimental import pallas as pl
from jax.experimental.pallas import tpu as pltpu
from jax.experimental.pallas import tpu_sc as plsc

KK = 8192
DD = 32
NN = 8192
BN = 256
NB = NN // BN

NUM_WORKERS = 32
CH = NN // NUM_WORKERS


def _argmin_body(x_ref, w_ref, hsw_ref, tok_ref):
    x = x_ref[...]
    w = w_ref[...]
    hsw = hsw_ref[...]
    mm = lax.dot_general(x, w, (((1,), (1,)), ((), ())),
                         preferred_element_type=jnp.float32)
    m = hsw - mm
    minv = jnp.min(m, axis=1, keepdims=True)
    ii = lax.broadcasted_iota(jnp.int32, (BN, KK), 1)
    idx = jnp.min(jnp.where(m == minv, ii, jnp.int32(KK)), axis=1)
    tok_ref[0, 0, :] = idx


def _tokens(zf, w, hsw):
    tok3 = pl.pallas_call(
        _argmin_body,
        grid=(NB,),
        in_specs=[
            pl.BlockSpec((BN, DD), lambda i: (i, 0)),
            pl.BlockSpec((KK, DD), lambda i: (0, 0)),
            pl.BlockSpec((1, KK), lambda i: (0, 0)),
        ],
        out_specs=pl.BlockSpec((1, 1, BN), lambda i: (i, 0, 0)),
        out_shape=jax.ShapeDtypeStruct((NB, 1, BN), jnp.int32),
    )(zf, w, hsw)
    return tok3.reshape(NN)


def _fill(ref, rows, value):
    def body(i, carry):
        ref[i, :] = jnp.full((16,), value, jnp.float32)
        return carry
    lax.fori_loop(0, rows, body, 0)


def _sc_body(emb_hbm, tok_hbm, zq_hbm, hist_hbm,
             idx_v, rows_v, const_v, shared_hist, sem):
    cid = lax.axis_index("c")
    sid = lax.axis_index("s")
    wid = sid * 2 + cid
    base = wid * CH
    pltpu.sync_copy(tok_hbm.at[pl.ds(base, CH)], idx_v)
    pltpu.async_copy(emb_hbm.at[idx_v], rows_v, sem).wait()
    pltpu.sync_copy(rows_v, zq_hbm.at[pl.ds(base, CH)])
    zbase = sid * (KK // 16)
    _fill(const_v, CH, 0.0)
    pltpu.sync_copy(const_v, shared_hist.at[pl.ds(zbase, CH)])
    pltpu.sync_copy(const_v, shared_hist.at[pl.ds(zbase + CH, CH)])
    plsc.subcore_barrier()
    _fill(const_v, CH, 1.0)
    pltpu.sync_copy(const_v, shared_hist.at[idx_v], add=True)
    plsc.subcore_barrier()
    pltpu.sync_copy(shared_hist.at[pl.ds(zbase, CH)],
                    hist_hbm.at[cid, pl.ds(zbase, CH)])
    pltpu.sync_copy(shared_hist.at[pl.ds(zbase + CH, CH)],
                    hist_hbm.at[cid, pl.ds(zbase + CH, CH)])


def _gather_hist(emb, token):
    mesh = plsc.VectorSubcoreMesh(core_axis_name="c", subcore_axis_name="s")
    f = functools.partial(
        pl.kernel,
        mesh=mesh,
        compiler_params=pltpu.CompilerParams(use_tc_tiling_on_sc=False),
        out_type=[
            jax.ShapeDtypeStruct((NN, DD), jnp.float32),
            jax.ShapeDtypeStruct((2, KK, 16), jnp.float32),
        ],
        scratch_types=[
            pltpu.VMEM((CH,), jnp.int32),
            pltpu.VMEM((CH, DD), jnp.float32),
            pltpu.VMEM((CH, 16), jnp.float32),
            pltpu.VMEM_SHARED((KK, 16), jnp.float32),
            pltpu.SemaphoreType.DMA,
        ],
    )(_sc_body)
    return f(emb, token)


def _stats_body(zf_ref, zq_ref, hist_ref, st_ref,
                loss_ref, qe_ref, util_ref, perp_ref):
    zf = zf_ref[...]
    zq = zq_ref[...]
    dsq = (zq - zf) ** 2
    s = jnp.sum(dsq)
    m = s / jnp.float32(NN * DD)
    loss_ref[...] = jnp.reshape(0.25 * m + m, (1, 1))
    qe_ref[...] = jnp.reshape(s / jnp.float32(NN), (1, 1))
    h = hist_ref[0, :, 0:1] + hist_ref[1, :, 0:1]
    util_ref[...] = jnp.reshape(
        jnp.sum((h > 0).astype(jnp.float32)) / jnp.float32(KK), (1, 1))
    p = h / jnp.sum(h)
    perp_ref[...] = jnp.reshape(
        jnp.exp(-jnp.sum(p * jnp.log(p + 1e-10))), (1, 1))
    st_ref[...] = zf + (zq - zf)


def _stats(zf, zq, hist):
    return pl.pallas_call(
        _stats_body,
        out_shape=[
            jax.ShapeDtypeStruct((NN, DD), jnp.float32),
            jax.ShapeDtypeStruct((1, 1), jnp.float32),
            jax.ShapeDtypeStruct((1, 1), jnp.float32),
            jax.ShapeDtypeStruct((1, 1), jnp.float32),
            jax.ShapeDtypeStruct((1, 1), jnp.float32),
        ],
    )(zf, zq, hist)


def kernel(z, emb_weight):
    zp = jnp.transpose(z, (0, 2, 3, 1))
    zf = zp.reshape(NN, DD)
    hsw = 0.5 * jnp.sum(emb_weight ** 2, axis=1)
    token = _tokens(zf, emb_weight, hsw.reshape(1, KK))
    zq, hist = _gather_hist(emb_weight, token)
    st, loss, qe, util, perp = _stats(zf, zq, hist)
    out = jnp.transpose(st.reshape(zp.shape), (0, 3, 1, 2))
    return (out, loss[0, 0], qe[0, 0], util[0, 0], perp[0, 0])


# speedup vs baseline: 1.4193x; 1.3283x over previous
import functools

import jax
import jax.numpy as jnp
from jax import lax
from jax.experimental import pallas as pl
from jax.experimental.pallas import tpu as pltpu
from jax.experimental.pallas import tpu_sc as plsc

KK = 8192
DD = 32
NN = 8192
BN = 256
NB = NN // BN

NUM_WORKERS = 32
CH = NN // NUM_WORKERS


def _argmin_body(x_ref, w_ref, hsw_ref, tok_ref):
    x = x_ref[...]
    w = w_ref[...]
    hsw = hsw_ref[...]
    mm = lax.dot_general(x, w, (((1,), (1,)), ((), ())),
                         preferred_element_type=jnp.float32)
    m = hsw - mm
    idx = jnp.argmin(m, axis=1).astype(jnp.int32)
    tok_ref[0, 0, :] = idx


def _tokens(zf, w, hsw):
    tok3 = pl.pallas_call(
        _argmin_body,
        grid=(NB,),
        in_specs=[
            pl.BlockSpec((BN, DD), lambda i: (i, 0)),
            pl.BlockSpec((KK, DD), lambda i: (0, 0)),
            pl.BlockSpec((1, KK), lambda i: (0, 0)),
        ],
        out_specs=pl.BlockSpec((1, 1, BN), lambda i: (i, 0, 0)),
        out_shape=jax.ShapeDtypeStruct((NB, 1, BN), jnp.int32),
    )(zf, w, hsw)
    return tok3.reshape(NN)


def _fill(ref, rows, value):
    def body(i, carry):
        ref[i, :] = jnp.full((16,), value, jnp.float32)
        return carry
    lax.fori_loop(0, rows, body, 0)


def _sc_body(emb_hbm, tok_hbm, zq_hbm, hist_hbm,
             idx_v, rows_v, const_v, shared_hist, sem):
    cid = lax.axis_index("c")
    sid = lax.axis_index("s")
    wid = sid * 2 + cid
    base = wid * CH
    pltpu.sync_copy(tok_hbm.at[pl.ds(base, CH)], idx_v)
    pltpu.async_copy(emb_hbm.at[idx_v], rows_v, sem).wait()
    pltpu.sync_copy(rows_v, zq_hbm.at[pl.ds(base, CH)])
    zbase = sid * (KK // 16)
    _fill(const_v, CH, 0.0)
    pltpu.sync_copy(const_v, shared_hist.at[pl.ds(zbase, CH)])
    pltpu.sync_copy(const_v, shared_hist.at[pl.ds(zbase + CH, CH)])
    plsc.subcore_barrier()
    _fill(const_v, CH, 1.0)
    pltpu.sync_copy(const_v, shared_hist.at[idx_v], add=True)
    plsc.subcore_barrier()
    pltpu.sync_copy(shared_hist.at[pl.ds(zbase, CH)],
                    hist_hbm.at[cid, pl.ds(zbase, CH)])
    pltpu.sync_copy(shared_hist.at[pl.ds(zbase + CH, CH)],
                    hist_hbm.at[cid, pl.ds(zbase + CH, CH)])


def _gather_hist(emb, token):
    mesh = plsc.VectorSubcoreMesh(core_axis_name="c", subcore_axis_name="s")
    f = functools.partial(
        pl.kernel,
        mesh=mesh,
        compiler_params=pltpu.CompilerParams(use_tc_tiling_on_sc=False),
        out_type=[
            jax.ShapeDtypeStruct((NN, DD), jnp.float32),
            jax.ShapeDtypeStruct((2, KK, 16), jnp.float32),
        ],
        scratch_types=[
            pltpu.VMEM((CH,), jnp.int32),
            pltpu.VMEM((CH, DD), jnp.float32),
            pltpu.VMEM((CH, 16), jnp.float32),
            pltpu.VMEM_SHARED((KK, 16), jnp.float32),
            pltpu.SemaphoreType.DMA,
        ],
    )(_sc_body)
    return f(emb, token)


def _stats_body(zf_ref, zq_ref, hist_ref, st_ref,
                loss_ref, qe_ref, util_ref, perp_ref):
    zf = zf_ref[...]
    zq = zq_ref[...]
    dsq = (zq - zf) ** 2
    s = jnp.sum(dsq)
    m = s / jnp.float32(NN * DD)
    loss_ref[...] = jnp.reshape(0.25 * m + m, (1, 1))
    qe_ref[...] = jnp.reshape(s / jnp.float32(NN), (1, 1))
    h = hist_ref[0, :, 0:1] + hist_ref[1, :, 0:1]
    util_ref[...] = jnp.reshape(
        jnp.sum((h > 0).astype(jnp.float32)) / jnp.float32(KK), (1, 1))
    p = h / jnp.sum(h)
    perp_ref[...] = jnp.reshape(
        jnp.exp(-jnp.sum(p * jnp.log(p + 1e-10))), (1, 1))
    st_ref[...] = zf + (zq - zf)


def _stats(zf, zq, hist):
    return pl.pallas_call(
        _stats_body,
        out_shape=[
            jax.ShapeDtypeStruct((NN, DD), jnp.float32),
            jax.ShapeDtypeStruct((1, 1), jnp.float32),
            jax.ShapeDtypeStruct((1, 1), jnp.float32),
            jax.ShapeDtypeStruct((1, 1), jnp.float32),
            jax.ShapeDtypeStruct((1, 1), jnp.float32),
        ],
    )(zf, zq, hist)


def kernel(z, emb_weight):
    zp = jnp.transpose(z, (0, 2, 3, 1))
    zf = zp.reshape(NN, DD)
    hsw = 0.5 * jnp.sum(emb_weight ** 2, axis=1)
    token = _tokens(zf, emb_weight, hsw.reshape(1, KK))
    zq, hist = _gather_hist(emb_weight, token)
    st, loss, qe, util, perp = _stats(zf, zq, hist)
    out = jnp.transpose(st.reshape(zp.shape), (0, 3, 1, 2))
    return (out, loss[0, 0], qe[0, 0], util[0, 0], perp[0, 0])


# speedup vs baseline: 1.4685x; 1.0347x over previous
import functools

import jax
import jax.numpy as jnp
from jax import lax
from jax.experimental import pallas as pl
from jax.experimental.pallas import tpu as pltpu
from jax.experimental.pallas import tpu_sc as plsc

KK = 8192
DD = 32
NN = 8192
BN = 512
NB = NN // BN

NUM_WORKERS = 32
CH = NN // NUM_WORKERS


def _argmin_body(x_ref, w_ref, hsw_ref, tok_ref):
    x = x_ref[...]
    w = w_ref[...]
    hsw = hsw_ref[...]
    mm = lax.dot_general(x, w, (((1,), (1,)), ((), ())),
                         preferred_element_type=jnp.float32)
    m = hsw - mm
    idx = jnp.argmin(m, axis=1).astype(jnp.int32)
    tok_ref[0, 0, :] = idx


def _tokens(zf, w, hsw):
    tok3 = pl.pallas_call(
        _argmin_body,
        grid=(NB,),
        in_specs=[
            pl.BlockSpec((BN, DD), lambda i: (i, 0)),
            pl.BlockSpec((KK, DD), lambda i: (0, 0)),
            pl.BlockSpec((1, KK), lambda i: (0, 0)),
        ],
        out_specs=pl.BlockSpec((1, 1, BN), lambda i: (i, 0, 0)),
        out_shape=jax.ShapeDtypeStruct((NB, 1, BN), jnp.int32),
    )(zf, w, hsw)
    return tok3.reshape(NN)


def _fill(ref, rows, value):
    def body(i, carry):
        ref[i, :] = jnp.full((16,), value, jnp.float32)
        return carry
    lax.fori_loop(0, rows, body, 0)


def _sc_body(emb_hbm, tok_hbm, zq_hbm, hist_hbm,
             idx_v, rows_v, const_v, shared_hist, sem):
    cid = lax.axis_index("c")
    sid = lax.axis_index("s")
    wid = sid * 2 + cid
    base = wid * CH
    pltpu.sync_copy(tok_hbm.at[pl.ds(base, CH)], idx_v)
    pltpu.async_copy(emb_hbm.at[idx_v], rows_v, sem).wait()
    pltpu.sync_copy(rows_v, zq_hbm.at[pl.ds(base, CH)])
    zbase = sid * (KK // 16)
    _fill(const_v, CH, 0.0)
    pltpu.sync_copy(const_v, shared_hist.at[pl.ds(zbase, CH)])
    pltpu.sync_copy(const_v, shared_hist.at[pl.ds(zbase + CH, CH)])
    plsc.subcore_barrier()
    _fill(const_v, CH, 1.0)
    pltpu.sync_copy(const_v, shared_hist.at[idx_v], add=True)
    plsc.subcore_barrier()
    pltpu.sync_copy(shared_hist.at[pl.ds(zbase, CH)],
                    hist_hbm.at[cid, pl.ds(zbase, CH)])
    pltpu.sync_copy(shared_hist.at[pl.ds(zbase + CH, CH)],
                    hist_hbm.at[cid, pl.ds(zbase + CH, CH)])


def _gather_hist(emb, token):
    mesh = plsc.VectorSubcoreMesh(core_axis_name="c", subcore_axis_name="s")
    f = functools.partial(
        pl.kernel,
        mesh=mesh,
        compiler_params=pltpu.CompilerParams(use_tc_tiling_on_sc=False),
        out_type=[
            jax.ShapeDtypeStruct((NN, DD), jnp.float32),
            jax.ShapeDtypeStruct((2, KK, 16), jnp.float32),
        ],
        scratch_types=[
            pltpu.VMEM((CH,), jnp.int32),
            pltpu.VMEM((CH, DD), jnp.float32),
            pltpu.VMEM((CH, 16), jnp.float32),
            pltpu.VMEM_SHARED((KK, 16), jnp.float32),
            pltpu.SemaphoreType.DMA,
        ],
    )(_sc_body)
    return f(emb, token)


def _stats_body(zf_ref, zq_ref, hist_ref, st_ref,
                loss_ref, qe_ref, util_ref, perp_ref):
    zf = zf_ref[...]
    zq = zq_ref[...]
    dsq = (zq - zf) ** 2
    s = jnp.sum(dsq)
    m = s / jnp.float32(NN * DD)
    loss_ref[...] = jnp.reshape(0.25 * m + m, (1, 1))
    qe_ref[...] = jnp.reshape(s / jnp.float32(NN), (1, 1))
    h = hist_ref[0, :, 0:1] + hist_ref[1, :, 0:1]
    util_ref[...] = jnp.reshape(
        jnp.sum((h > 0).astype(jnp.float32)) / jnp.float32(KK), (1, 1))
    p = h / jnp.sum(h)
    perp_ref[...] = jnp.reshape(
        jnp.exp(-jnp.sum(p * jnp.log(p + 1e-10))), (1, 1))
    st_ref[...] = zf + (zq - zf)


def _stats(zf, zq, hist):
    return pl.pallas_call(
        _stats_body,
        out_shape=[
            jax.ShapeDtypeStruct((NN, DD), jnp.float32),
            jax.ShapeDtypeStruct((1, 1), jnp.float32),
            jax.ShapeDtypeStruct((1, 1), jnp.float32),
            jax.ShapeDtypeStruct((1, 1), jnp.float32),
            jax.ShapeDtypeStruct((1, 1), jnp.float32),
        ],
    )(zf, zq, hist)


def kernel(z, emb_weight):
    zp = jnp.transpose(z, (0, 2, 3, 1))
    zf = zp.reshape(NN, DD)
    hsw = 0.5 * jnp.sum(emb_weight ** 2, axis=1)
    token = _tokens(zf, emb_weight, hsw.reshape(1, KK))
    zq, hist = _gather_hist(emb_weight, token)
    st, loss, qe, util, perp = _stats(zf, zq, hist)
    out = jnp.transpose(st.reshape(zp.shape), (0, 3, 1, 2))
    return (out, loss[0, 0], qe[0, 0], util[0, 0], perp[0, 0])
